# baseline (speedup 1.0000x reference)
import functools

import jax
import jax.numpy as jnp
from jax import lax
from jax.experimental import pallas as pl
from jax.experimental.pallas import tpu as pltpu
from jax.experimental.pallas import tpu_sc as plsc

NC = 2
NS = 16
NW = NC * NS
LANES = 16
CH = 32
ZR = 32
SB = 32
NB = 8
LG = 7


def _mesh():
    return plsc.VectorSubcoreMesh(core_axis_name="c", subcore_axis_name="s")


DCH = 80


def _build_deg(E_pad, N_pad):
    nch = (E_pad // DCH) // NW
    assert E_pad % (DCH * NW * 8) == 0 and DCH % LANES == 0

    @functools.partial(
        pl.kernel,
        out_type=jax.ShapeDtypeStruct((NC, N_pad), jnp.float32),
        mesh=_mesh(),
        scratch_types=[
            pltpu.VMEM((nch, DCH), jnp.int32),
            pltpu.VMEM((DCH,), jnp.float32),
            pltpu.VMEM((N_pad,), jnp.float32),
            pltpu.VMEM_SHARED((N_pad,), jnp.float32),
            pltpu.SemaphoreType.DMA,
        ],
    )
    def deg_kernel(col_hbm, out_hbm, colbuf, ones_v, stage, deg_sh, dsem):
        c = lax.axis_index("c")
        s = lax.axis_index("s")
        wid = s * NC + c
        z16 = jnp.zeros((LANES,), jnp.float32)
        o16 = jnp.ones((LANES,), jnp.float32)
        for k in range(DCH // LANES):
            ones_v[pl.ds(k * LANES, LANES)] = o16

        def zb(i, carry):
            stage[pl.ds(i * LANES, LANES)] = z16
            return carry

        lax.fori_loop(0, N_pad // LANES, zb, 0)

        @pl.when(s == 0)
        def _():
            pltpu.sync_copy(stage, deg_sh)

        plsc.subcore_barrier()
        pltpu.sync_copy(col_hbm.at[pl.ds(wid * nch, nch)], colbuf)

        def eb(g, carry):
            for t in range(8):
                pltpu.async_copy(ones_v, deg_sh.at[colbuf.at[8 * g + t]],
                                 dsem, add=True)
            for t in range(8):
                pltpu.make_async_copy(ones_v, deg_sh.at[colbuf.at[8 * g + t]],
                                      dsem).wait()
            return carry

        lax.fori_loop(0, nch // 8, eb, 0)
        plsc.subcore_barrier()

        @pl.when(s == 0)
        def _():
            pltpu.sync_copy(deg_sh, stage)
            pltpu.sync_copy(stage, out_hbm.at[c])

    return deg_kernel


def _build_spmm(E_pad, N_pad, D):
    nch = (E_pad // CH) // NW
    rpt = N_pad // NS
    nsb = nch // SB
    assert nch % SB == 0 and SB % 2 == 0 and ZR <= CH and rpt % ZR == 0
    assert nsb >= 2 and SB >= 8

    @functools.partial(
        pl.kernel,
        out_type=jax.ShapeDtypeStruct((NC, N_pad, D), jnp.float32),
        mesh=_mesh(),
        scratch_types=[
            pltpu.VMEM((2 * SB, CH), jnp.int32),
            pltpu.VMEM((2 * SB, CH), jnp.int32),
            pltpu.VMEM((NB, CH, D), jnp.bfloat16),
            pltpu.VMEM((2, CH, D), jnp.float32),
            pltpu.VMEM_SHARED((N_pad, D), jnp.float32),
            pltpu.SemaphoreType.DMA((NB,)),
            pltpu.SemaphoreType.DMA((2,)),
            pltpu.SemaphoreType.DMA,
            pltpu.SemaphoreType.DMA,
            pltpu.SemaphoreType.DMA,
        ],
        compiler_params=pltpu.CompilerParams(use_tc_tiling_on_sc=False,
                                             needs_layout_passes=False),
    )
    def spmm_kernel(row_hbm, col_hbm, y_hbm, out_hbm,
                    rowbuf, colbuf, gbuf, fbuf, agg_sh, gsem, ssem, irsem,
                    icsem, zsem):
        c = lax.axis_index("c")
        s = lax.axis_index("s")
        wid = s * NC + c
        z16 = jnp.zeros((LANES,), jnp.float32)

        def zb(i, carry):
            for k in range(D // LANES):
                fbuf[0, i, pl.ds(k * LANES, LANES)] = z16
            return carry

        lax.fori_loop(0, ZR, zb, 0)
        for t in range(rpt // ZR):
            pltpu.async_copy(fbuf.at[0].at[pl.ds(0, ZR)],
                             agg_sh.at[pl.ds(s * rpt + t * ZR, ZR)], zsem)

        def gather(j, k):
            pltpu.async_copy(y_hbm.at[rowbuf.at[j % (2 * SB)]],
                             gbuf.at[k], gsem.at[k])

        def wait_gather(j, k):
            pltpu.make_async_copy(y_hbm.at[rowbuf.at[j % (2 * SB)]],
                                  gbuf.at[k], gsem.at[k]).wait()

        def scatter(j, p):
            pltpu.async_copy(fbuf.at[p], agg_sh.at[colbuf.at[j % (2 * SB)]],
                             ssem.at[p], add=True)

        def wait_scatter(j, p):
            pltpu.make_async_copy(fbuf.at[p],
                                  agg_sh.at[colbuf.at[j % (2 * SB)]],
                                  ssem.at[p]).wait()

        def widen(k, p):
            for r in range(CH):
                for q in range(D // 32):
                    x = gbuf[k, r, pl.ds(32 * q, 32)]
                    a, b = plsc.unpack(x, format=plsc.PackFormat.INTERLEAVED,
                                       preferred_element_type=jnp.float32)
                    fbuf[p, r, pl.ds(32 * q, LANES)] = a
                    fbuf[p, r, pl.ds(32 * q + LANES, LANES)] = b

        base0 = wid * nch
        pltpu.sync_copy(row_hbm.at[pl.ds(base0, SB)], rowbuf.at[pl.ds(0, SB)])
        pltpu.sync_copy(col_hbm.at[pl.ds(base0, SB)], colbuf.at[pl.ds(0, SB)])
        for t in range(LG):
            gather(t, t)
        for t in range(rpt // ZR):
            pltpu.make_async_copy(
                fbuf.at[0].at[pl.ds(0, ZR)],
                agg_sh.at[pl.ds(s * rpt + t * ZR, ZR)], zsem).wait()
        plsc.subcore_barrier()

        def eb(j, carry):
            k = j % NB
            p = j % 2
            wait_gather(j, k)

            @pl.when(j >= 2)
            def _():
                wait_scatter(j - 2, p)

            widen(k, p)
            scatter(j, p)

            @pl.when(j + LG < nch)
            def _():
                k2 = (j + LG) % NB

                @pl.when((j + LG) % SB == 0)
                def _():
                    pltpu.make_async_copy(
                        row_hbm.at[pl.ds(base0, SB)],
                        rowbuf.at[pl.ds(0, SB)], irsem).wait()

                @pl.when((j + LG) % SB == 1)
                def _():
                    pltpu.make_async_copy(
                        col_hbm.at[pl.ds(base0, SB)],
                        colbuf.at[pl.ds(0, SB)], icsem).wait()

                gather(j + LG, k2)

            @pl.when((j % SB == NB - LG) & (j // SB + 1 < nsb))
            def _():
                nxt = j // SB + 1
                base = base0 + nxt * SB
                half = (nxt % 2) * SB
                pltpu.async_copy(row_hbm.at[pl.ds(base, SB)],
                                 rowbuf.at[pl.ds(half, SB)], irsem)
                pltpu.async_copy(col_hbm.at[pl.ds(base, SB)],
                                 colbuf.at[pl.ds(half, SB)], icsem)

            return carry

        lax.fori_loop(0, nch, eb, 0)
        for t in range(2):
            j = nch - 2 + t
            wait_scatter(j, j % 2)
        plsc.subcore_barrier()

        for t in range(rpt // ZR):
            sl = pl.ds(s * rpt + t * ZR, ZR)
            pltpu.async_copy(agg_sh.at[sl], out_hbm.at[c].at[sl], zsem)
        for t in range(rpt // ZR):
            sl = pl.ds(s * rpt + t * ZR, ZR)
            pltpu.make_async_copy(agg_sh.at[sl], out_hbm.at[c].at[sl],
                                  zsem).wait()

    return spmm_kernel


def _dinv(dp, N):
    deg = dp[0, :N] + dp[1, :N]
    return jnp.where(deg > 0, lax.rsqrt(deg), 0.0)


def _feature_perm(D):
    pos = jnp.arange(D)
    q = pos // 32
    r = pos % 32
    perm = jnp.where(r % 2 == 0, 32 * q + r // 2, 32 * q + 16 + r // 2)
    return jnp.zeros((D, D), jnp.float32).at[perm, jnp.arange(D)].set(1.0)


def _tc_prep(dp3, x, mf):
    N, D = x.shape

    def body(dp_ref, x_ref, mf_ref, p_ref, y_ref):
        dinv = _dinv(dp_ref[...], N)
        y = dinv * (mf_ref[...] * x_ref[...])
        y_ref[...] = lax.dot_general(
            y, p_ref[...], (((1,), (0,)), ((), ())),
            preferred_element_type=jnp.float32).astype(jnp.bfloat16)

    return pl.pallas_call(
        body, out_shape=jax.ShapeDtypeStruct((N, D), jnp.bfloat16),
    )(dp3, x, mf, _feature_perm(D))


def _tc_layer(ap, dp3, x, mf, W, b, bias):
    N, D = x.shape

    def body(ap_ref, dp_ref, x_ref, mf_ref, w_ref, b_ref, bias_ref, p_ref,
             y_ref):
        dinv = _dinv(dp_ref[...], N)
        ap = ap_ref[...]
        agg = ap[0, :N] + ap[1, :N]
        t = agg * dinv
        h = lax.dot_general(t, w_ref[...], (((1,), (1,)), ((), ())),
                            preferred_element_type=jnp.float32)
        h = jnp.maximum(h + b_ref[...] + bias_ref[...], 0.0)
        mfv = mf_ref[...]
        y = dinv * (mfv * x_ref[...] + (1.0 - mfv) * h)
        y_ref[...] = lax.dot_general(
            y, p_ref[...], (((1,), (0,)), ((), ())),
            preferred_element_type=jnp.float32).astype(jnp.bfloat16)

    return pl.pallas_call(
        body, out_shape=jax.ShapeDtypeStruct((N, D), jnp.bfloat16),
    )(ap, dp3, x, mf, W, b, bias, _feature_perm(D))


def _tc_final(ap, dp3, W, b, bias, fcW, fcb, N):
    D = ap.shape[2]

    def body(ap_ref, dp_ref, w_ref, b_ref, bias_ref, fw_ref, fb_ref, o_ref):
        dinv = _dinv(dp_ref[...], N)
        ap_v = ap_ref[...]
        agg = ap_v[0, :N] + ap_v[1, :N]
        t = agg * dinv
        h = lax.dot_general(t, w_ref[...], (((1,), (1,)), ((), ())),
                            preferred_element_type=jnp.float32)
        h = jnp.maximum(h + b_ref[...] + bias_ref[...], 0.0)
        o_ref[...] = lax.dot_general(h, fw_ref[...], (((1,), (1,)), ((), ())),
                                     preferred_element_type=jnp.float32) + fb_ref[...]

    return pl.pallas_call(
        body, out_shape=jax.ShapeDtypeStruct((N, D), jnp.float32),
    )(ap, dp3, W, b, bias, fcW, fcb)


def kernel(edge_index, edge_weight, x, M,
           W1, b1, bias1, W2, b2, bias2, W3, b3, bias3, fcW, fcb):
    del edge_weight
    N, D = x.shape
    E = edge_index.shape[1]

    blk = CH * NW * 8
    E_pad = ((E + blk - 1) // blk) * blk
    N_pad = ((N + NS * ZR - 1) // (NS * ZR)) * (NS * ZR)
    rpt = N_pad // NS
    assert rpt % ZR == 0 and D % LANES == 0 and N_pad > N

    pad = E_pad - E
    padi = jnp.arange(pad, dtype=jnp.int32)
    rowp = jnp.concatenate([edge_index[0], (padi * 997) % N])
    colp = jnp.concatenate([edge_index[1], N + padi % (N_pad - N)])
    row2 = rowp.reshape(E_pad // CH, CH)
    col2 = colp.reshape(E_pad // CH, CH)
    mf = M.astype(jnp.float32)

    deg_parts = _build_deg(E_pad, N_pad)(colp.reshape(E_pad // DCH, DCH))
    dp3 = deg_parts.reshape(NC, N_pad, 1)

    spmm = _build_spmm(E_pad, N_pad, D)
    b1r, bias1r = b1.reshape(1, D), bias1.reshape(1, D)
    b2r, bias2r = b2.reshape(1, D), bias2.reshape(1, D)
    b3r, bias3r = b3.reshape(1, D), bias3.reshape(1, D)
    fcbr = fcb.reshape(1, D)

    y = _tc_prep(dp3, x, mf)
    ap = spmm(row2, col2, y)
    y = _tc_layer(ap, dp3, x, mf, W1, b1r, bias1r)
    ap = spmm(row2, col2, y)
    y = _tc_layer(ap, dp3, x, mf, W2, b2r, bias2r)
    ap = spmm(row2, col2, y)
    return _tc_final(ap, dp3, W3, b3r, bias3r, fcW, fcbr, N)


# speedup vs baseline: 1.9999x; 1.9999x over previous
import functools

import jax
import jax.numpy as jnp
from jax import lax
from jax.experimental import pallas as pl
from jax.experimental.pallas import tpu as pltpu
from jax.experimental.pallas import tpu_sc as plsc

NC = 2
NS = 16
NW = NC * NS
LANES = 16
CH = 32
ZR = 32
SB = 32
NB = 8
LG = 7


def _mesh():
    return plsc.VectorSubcoreMesh(core_axis_name="c", subcore_axis_name="s")


DCH = 80


def _build_deg(E_pad, N_pad):
    nch = (E_pad // DCH) // NW
    assert E_pad % (DCH * NW * 8) == 0 and DCH % LANES == 0

    @functools.partial(
        pl.kernel,
        out_type=jax.ShapeDtypeStruct((NC, N_pad), jnp.float32),
        mesh=_mesh(),
        scratch_types=[
            pltpu.VMEM((nch, DCH), jnp.int32),
            pltpu.VMEM((DCH,), jnp.float32),
            pltpu.VMEM((N_pad,), jnp.float32),
            pltpu.VMEM_SHARED((N_pad,), jnp.float32),
            pltpu.SemaphoreType.DMA,
        ],
    )
    def deg_kernel(col_hbm, out_hbm, colbuf, ones_v, stage, deg_sh, dsem):
        c = lax.axis_index("c")
        s = lax.axis_index("s")
        wid = s * NC + c
        z16 = jnp.zeros((LANES,), jnp.float32)
        o16 = jnp.ones((LANES,), jnp.float32)
        for k in range(DCH // LANES):
            ones_v[pl.ds(k * LANES, LANES)] = o16

        def zb(i, carry):
            stage[pl.ds(i * LANES, LANES)] = z16
            return carry

        lax.fori_loop(0, N_pad // LANES, zb, 0)

        @pl.when(s == 0)
        def _():
            pltpu.sync_copy(stage, deg_sh)

        plsc.subcore_barrier()
        pltpu.sync_copy(col_hbm.at[pl.ds(wid * nch, nch)], colbuf)

        def eb(g, carry):
            for t in range(8):
                pltpu.async_copy(ones_v, deg_sh.at[colbuf.at[8 * g + t]],
                                 dsem, add=True)
            for t in range(8):
                pltpu.make_async_copy(ones_v, deg_sh.at[colbuf.at[8 * g + t]],
                                      dsem).wait()
            return carry

        lax.fori_loop(0, nch // 8, eb, 0)
        plsc.subcore_barrier()

        @pl.when(s == 0)
        def _():
            pltpu.sync_copy(deg_sh, stage)
            pltpu.sync_copy(stage, out_hbm.at[c])

    return deg_kernel


def _build_spmm(E_pad, N_pad, D):
    nch = (E_pad // CH) // NW
    rpt = N_pad // NS
    nsb = nch // SB
    assert nch % SB == 0 and SB % 2 == 0 and ZR <= CH and rpt % ZR == 0
    assert nsb >= 2 and SB >= 8

    @functools.partial(
        pl.kernel,
        out_type=jax.ShapeDtypeStruct((NC, N_pad, D), jnp.float32),
        mesh=_mesh(),
        scratch_types=[
            pltpu.VMEM((2 * SB, CH), jnp.int32),
            pltpu.VMEM((2 * SB, CH), jnp.int32),
            pltpu.VMEM((NB, CH, D), jnp.float32),
            pltpu.VMEM_SHARED((N_pad, D), jnp.float32),
            pltpu.SemaphoreType.DMA((NB,)),
            pltpu.SemaphoreType.DMA((NB,)),
            pltpu.SemaphoreType.DMA,
            pltpu.SemaphoreType.DMA,
            pltpu.SemaphoreType.DMA,
        ],
    )
    def spmm_kernel(row_hbm, col_hbm, y_hbm, out_hbm,
                    rowbuf, colbuf, gbuf, agg_sh, gsem, ssem, irsem, icsem,
                    zsem):
        c = lax.axis_index("c")
        s = lax.axis_index("s")
        wid = s * NC + c
        z16 = jnp.zeros((LANES,), jnp.float32)

        def zb(i, carry):
            for k in range(D // LANES):
                gbuf[0, i, pl.ds(k * LANES, LANES)] = z16
            return carry

        lax.fori_loop(0, ZR, zb, 0)
        for t in range(rpt // ZR):
            pltpu.async_copy(gbuf.at[0].at[pl.ds(0, ZR)],
                             agg_sh.at[pl.ds(s * rpt + t * ZR, ZR)], zsem)

        def gather(j, k):
            pltpu.async_copy(y_hbm.at[rowbuf.at[j % (2 * SB)]],
                             gbuf.at[k], gsem.at[k])

        def wait_gather(j, k):
            pltpu.make_async_copy(y_hbm.at[rowbuf.at[j % (2 * SB)]],
                                  gbuf.at[k], gsem.at[k]).wait()

        def scatter(j, k):
            pltpu.async_copy(gbuf.at[k], agg_sh.at[colbuf.at[j % (2 * SB)]],
                             ssem.at[k], add=True)

        def wait_scatter(j, k):
            pltpu.make_async_copy(gbuf.at[k],
                                  agg_sh.at[colbuf.at[j % (2 * SB)]],
                                  ssem.at[k]).wait()

        base0 = wid * nch
        pltpu.sync_copy(row_hbm.at[pl.ds(base0, SB)], rowbuf.at[pl.ds(0, SB)])
        pltpu.sync_copy(col_hbm.at[pl.ds(base0, SB)], colbuf.at[pl.ds(0, SB)])
        for t in range(1, LG):
            gather(t, t)
        for t in range(rpt // ZR):
            pltpu.make_async_copy(
                gbuf.at[0].at[pl.ds(0, ZR)],
                agg_sh.at[pl.ds(s * rpt + t * ZR, ZR)], zsem).wait()
        plsc.subcore_barrier()
        gather(0, 0)

        def eb(j, carry):
            k = j % NB
            wait_gather(j, k)
            scatter(j, k)

            @pl.when(j + LG < nch)
            def _():
                k2 = (j + LG) % NB

                @pl.when(j >= NB - LG)
                def _():
                    wait_scatter(j - (NB - LG), k2)

                @pl.when((j + LG) % SB == 0)
                def _():
                    pltpu.make_async_copy(
                        row_hbm.at[pl.ds(base0, SB)],
                        rowbuf.at[pl.ds(0, SB)], irsem).wait()

                @pl.when((j + LG) % SB == 1)
                def _():
                    pltpu.make_async_copy(
                        col_hbm.at[pl.ds(base0, SB)],
                        colbuf.at[pl.ds(0, SB)], icsem).wait()

                gather(j + LG, k2)

            @pl.when((j % SB == NB - LG) & (j // SB + 1 < nsb))
            def _():
                nxt = j // SB + 1
                base = base0 + nxt * SB
                half = (nxt % 2) * SB
                pltpu.async_copy(row_hbm.at[pl.ds(base, SB)],
                                 rowbuf.at[pl.ds(half, SB)], irsem)
                pltpu.async_copy(col_hbm.at[pl.ds(base, SB)],
                                 colbuf.at[pl.ds(half, SB)], icsem)

            return carry

        lax.fori_loop(0, nch, eb, 0)
        for t in range(NB):
            j = nch - NB + t
            wait_scatter(j, j % NB)
        plsc.subcore_barrier()

        for t in range(rpt // ZR):
            sl = pl.ds(s * rpt + t * ZR, ZR)
            pltpu.async_copy(agg_sh.at[sl], out_hbm.at[c].at[sl], zsem)
        for t in range(rpt // ZR):
            sl = pl.ds(s * rpt + t * ZR, ZR)
            pltpu.make_async_copy(agg_sh.at[sl], out_hbm.at[c].at[sl],
                                  zsem).wait()

    return spmm_kernel


def _dinv(dp, N):
    deg = dp[0, :N] + dp[1, :N]
    return jnp.where(deg > 0, lax.rsqrt(deg), 0.0)


def _tc_prep(dp3, x, mf):
    N, D = x.shape

    def body(dp_ref, x_ref, mf_ref, y_ref):
        dinv = _dinv(dp_ref[...], N)
        y_ref[...] = dinv * (mf_ref[...] * x_ref[...])

    return pl.pallas_call(
        body, out_shape=jax.ShapeDtypeStruct((N, D), jnp.float32),
    )(dp3, x, mf)


def _tc_layer(ap, dp3, x, mf, W, b, bias):
    N, D = x.shape

    def body(ap_ref, dp_ref, x_ref, mf_ref, w_ref, b_ref, bias_ref, y_ref):
        dinv = _dinv(dp_ref[...], N)
        ap = ap_ref[...]
        agg = ap[0, :N] + ap[1, :N]
        t = agg * dinv
        h = lax.dot_general(t, w_ref[...], (((1,), (1,)), ((), ())),
                            preferred_element_type=jnp.float32)
        h = jnp.maximum(h + b_ref[...] + bias_ref[...], 0.0)
        mfv = mf_ref[...]
        y_ref[...] = dinv * (mfv * x_ref[...] + (1.0 - mfv) * h)

    return pl.pallas_call(
        body, out_shape=jax.ShapeDtypeStruct((N, D), jnp.float32),
    )(ap, dp3, x, mf, W, b, bias)


def _tc_final(ap, dp3, W, b, bias, fcW, fcb, N):
    D = ap.shape[2]

    def body(ap_ref, dp_ref, w_ref, b_ref, bias_ref, fw_ref, fb_ref, o_ref):
        dinv = _dinv(dp_ref[...], N)
        ap_v = ap_ref[...]
        agg = ap_v[0, :N] + ap_v[1, :N]
        t = agg * dinv
        h = lax.dot_general(t, w_ref[...], (((1,), (1,)), ((), ())),
                            preferred_element_type=jnp.float32)
        h = jnp.maximum(h + b_ref[...] + bias_ref[...], 0.0)
        o_ref[...] = lax.dot_general(h, fw_ref[...], (((1,), (1,)), ((), ())),
                                     preferred_element_type=jnp.float32) + fb_ref[...]

    return pl.pallas_call(
        body, out_shape=jax.ShapeDtypeStruct((N, D), jnp.float32),
    )(ap, dp3, W, b, bias, fcW, fcb)


def kernel(edge_index, edge_weight, x, M,
           W1, b1, bias1, W2, b2, bias2, W3, b3, bias3, fcW, fcb):
    del edge_weight
    N, D = x.shape
    E = edge_index.shape[1]

    blk = CH * NW * 8
    E_pad = ((E + blk - 1) // blk) * blk
    N_pad = ((N + NS * ZR - 1) // (NS * ZR)) * (NS * ZR)
    rpt = N_pad // NS
    assert rpt % ZR == 0 and D % LANES == 0 and N_pad > N

    pad = E_pad - E
    padi = jnp.arange(pad, dtype=jnp.int32)
    rowp = jnp.concatenate([edge_index[0], (padi * 997) % N])
    colp = jnp.concatenate([edge_index[1], N + padi % (N_pad - N)])
    row2 = rowp.reshape(E_pad // CH, CH)
    col2 = colp.reshape(E_pad // CH, CH)
    mf = M.astype(jnp.float32)

    deg_parts = _build_deg(E_pad, N_pad)(colp.reshape(E_pad // DCH, DCH))
    dp3 = deg_parts.reshape(NC, N_pad, 1)

    spmm = _build_spmm(E_pad, N_pad, D)
    b1r, bias1r = b1.reshape(1, D), bias1.reshape(1, D)
    b2r, bias2r = b2.reshape(1, D), bias2.reshape(1, D)
    b3r, bias3r = b3.reshape(1, D), bias3.reshape(1, D)
    fcbr = fcb.reshape(1, D)

    y = _tc_prep(dp3, x, mf)
    ap = spmm(row2, col2, y)
    y = _tc_layer(ap, dp3, x, mf, W1, b1r, bias1r)
    ap = spmm(row2, col2, y)
    y = _tc_layer(ap, dp3, x, mf, W2, b2r, bias2r)
    ap = spmm(row2, col2, y)
    return _tc_final(ap, dp3, W3, b3r, bias3r, fcW, fcbr, N)
